# baseline (speedup 1.0000x reference)
import functools

import jax
import jax.numpy as jnp
from jax import lax
from jax.experimental import pallas as pl
from jax.experimental.pallas import tpu as pltpu
from jax.experimental.pallas import tpu_sc as plsc

NUM_TIMESTEPS = 1000
EMBED_DIM = 128
BATCH = 16384

B_SC = 8192
B_TC = BATCH - B_SC

NC = 2
NS = 16
NW = NC * NS
B_PER_W = B_SC // NW
CHUNK = 128
NCHUNK = B_PER_W // CHUNK


@functools.partial(
    pl.kernel,
    mesh=plsc.VectorSubcoreMesh(core_axis_name="c", subcore_axis_name="s", num_cores=NC),
    out_type=jax.ShapeDtypeStruct((BATCH, EMBED_DIM), jnp.float32),
    scratch_types=[
        pltpu.VMEM((NCHUNK, CHUNK), jnp.int32),
        pltpu.VMEM((B_PER_W, EMBED_DIM), jnp.float32),
        pltpu.SemaphoreType.DMA,
    ],
)
def _sc_gather(ts_hbm, table_hbm, out_hbm, idx_v, rows_v, sem_g):
    wid = lax.axis_index("s") * NC + lax.axis_index("c")
    pltpu.sync_copy(ts_hbm.at[pl.ds(wid * NCHUNK, NCHUNK)], idx_v)
    gathers = [
        pltpu.async_copy(
            table_hbm.at[idx_v.at[j]],
            rows_v.at[pl.ds(j * CHUNK, CHUNK)],
            sem_g,
        )
        for j in range(NCHUNK)
    ]
    for c in gathers:
        c.wait()
    pltpu.sync_copy(rows_v, out_hbm.at[pl.ds(wid * B_PER_W, B_PER_W)])


V_PAD = 1024
TC_BLK = 1024


def _tc_body(ts_ref, tab_ref, o_ref):
    onehot = (
        ts_ref[...] == lax.broadcasted_iota(jnp.int32, (TC_BLK, V_PAD), 1)
    ).astype(jnp.bfloat16)
    o_ref[...] = jnp.dot(onehot, tab_ref[...], preferred_element_type=jnp.float32)


_tc_onehot = pl.pallas_call(
    _tc_body,
    grid=(B_TC // TC_BLK,),
    in_specs=[
        pl.BlockSpec((TC_BLK, 1), lambda i: (i, 0)),
        pl.BlockSpec((V_PAD, EMBED_DIM), lambda i: (0, 0)),
    ],
    out_specs=pl.BlockSpec((TC_BLK, EMBED_DIM), lambda i: (i, 0)),
    out_shape=jax.ShapeDtypeStruct((B_TC, EMBED_DIM), jnp.float32),
    compiler_params=pltpu.CompilerParams(
        dimension_semantics=("parallel",),
    ),
)


def kernel(timesteps, table):
    if timesteps.ndim == 2:
        timesteps = jnp.squeeze(timesteps, axis=1)
    ts = timesteps.astype(jnp.int32)
    ts_sc = ts[:B_SC].reshape(B_SC // CHUNK, CHUNK)
    ts_tc = ts[B_SC:].reshape(B_TC, 1)
    tab_p = jnp.zeros((V_PAD, EMBED_DIM), jnp.bfloat16).at[:NUM_TIMESTEPS].set(
        table.astype(jnp.bfloat16)
    )
    out_sc = _sc_gather(ts_sc, table)
    out_tc = _tc_onehot(ts_tc, tab_p)
    return lax.dynamic_update_slice(out_sc, out_tc, (B_SC, 0))


# speedup vs baseline: 1.2597x; 1.2597x over previous
import functools

import jax
import jax.numpy as jnp
from jax import lax
from jax.experimental import pallas as pl
from jax.experimental.pallas import tpu as pltpu
from jax.experimental.pallas import tpu_sc as plsc

NUM_TIMESTEPS = 1000
EMBED_DIM = 128
BATCH = 16384

NC = 2
NS = 16
NW = NC * NS
B_PER_W = BATCH // NW
CHUNK = 128
NCHUNK = B_PER_W // CHUNK


@functools.partial(
    pl.kernel,
    mesh=plsc.VectorSubcoreMesh(core_axis_name="c", subcore_axis_name="s", num_cores=NC),
    out_type=jax.ShapeDtypeStruct((BATCH, EMBED_DIM), jnp.float32),
    scratch_types=[
        pltpu.VMEM((NCHUNK, CHUNK), jnp.int32),
        pltpu.VMEM((B_PER_W, EMBED_DIM), jnp.float32),
        pltpu.SemaphoreType.DMA,
    ],
)
def _sc_gather(ts_hbm, table_hbm, out_hbm, idx_v, rows_v, sem_g):
    wid = lax.axis_index("s") * NC + lax.axis_index("c")
    pltpu.sync_copy(ts_hbm.at[pl.ds(wid * NCHUNK, NCHUNK)], idx_v)
    gathers = [
        pltpu.async_copy(
            table_hbm.at[idx_v.at[j]],
            rows_v.at[pl.ds(j * CHUNK, CHUNK)],
            sem_g,
        )
        for j in range(NCHUNK)
    ]
    for c in gathers:
        c.wait()
    pltpu.sync_copy(rows_v, out_hbm.at[pl.ds(wid * B_PER_W, B_PER_W)])


def kernel(timesteps, table):
    if timesteps.ndim == 2:
        timesteps = jnp.squeeze(timesteps, axis=1)
    ts2d = timesteps.astype(jnp.int32).reshape(BATCH // CHUNK, CHUNK)
    return _sc_gather(ts2d, table)
